# initial kernel scaffold (unmeasured)
import jax
import jax.numpy as jnp
from jax import lax
from jax.experimental import pallas as pl
from jax.experimental.pallas import tpu as pltpu

N_DEV = 32
R_STEPS = N_DEV // 2
L_STEPS = N_DEV // 2 - 1
HC_ROUNDS = 5


def kernel(x, w_mat):
    x = x.astype(jnp.bfloat16)
    w_mat = w_mat.astype(jnp.bfloat16)
    m_per, k = x.shape
    _, n_local = w_mat.shape
    m_total = N_DEV * m_per

    def body(x_ref, w_ref, out_ref, comm_ref,
             r_send, r_recv, l_send, l_recv,
             hc_buf, hc_in, hc_ss, hc_rs):
        my = lax.axis_index("i")
        right = lax.rem(my + 1, N_DEV)
        left = lax.rem(my + N_DEV - 1, N_DEV)

        barrier = pltpu.get_barrier_semaphore()
        for nbr in (left, right):
            pl.semaphore_signal(barrier, inc=1, device_id=(nbr,),
                                device_id_type=pl.DeviceIdType.MESH)
        pl.semaphore_wait(barrier, 2)

        comm_ref[0] = x_ref[...]

        def gemm(slot):
            origin = lax.rem(my - slot + N_DEV, N_DEV)
            out_ref[pl.ds(origin * m_per, m_per), :] = jnp.dot(
                comm_ref[slot], w_ref[...], preferred_element_type=jnp.float32)

        rdmas = []
        for r in range(1, R_STEPS + 1):
            rd = pltpu.make_async_remote_copy(
                src_ref=comm_ref.at[r - 1], dst_ref=comm_ref.at[r],
                send_sem=r_send.at[r - 1], recv_sem=r_recv.at[r - 1],
                device_id=(right,), device_id_type=pl.DeviceIdType.MESH)
            rd.start()
            rdmas.append(rd)
            ld = None
            if r <= L_STEPS:
                src_slot = 0 if r == 1 else N_DEV + 1 - r
                ld = pltpu.make_async_remote_copy(
                    src_ref=comm_ref.at[src_slot], dst_ref=comm_ref.at[N_DEV - r],
                    send_sem=l_send.at[r - 1], recv_sem=l_recv.at[r - 1],
                    device_id=(left,), device_id_type=pl.DeviceIdType.MESH)
                ld.start()
                rdmas.append(ld)
            if r == 1:
                gemm(0)
            else:
                gemm(r - 1)
                gemm(N_DEV + 1 - r)
            rd.wait_recv()
            if ld is not None:
                ld.wait_recv()
        gemm(R_STEPS)
        gemm(R_STEPS + 1)
        for rd in rdmas:
            rd.wait_send()

        cur = jnp.max(jnp.abs(out_ref[...]))
        for kk in range(HC_ROUNDS):
            partner = jnp.bitwise_xor(my, 1 << kk)
            hc_buf[0, 0] = cur
            rd = pltpu.make_async_remote_copy(
                src_ref=hc_buf, dst_ref=hc_in.at[kk],
                send_sem=hc_ss.at[kk], recv_sem=hc_rs.at[kk],
                device_id=(partner,), device_id_type=pl.DeviceIdType.MESH)
            rd.start()
            rd.wait()
            cur = jnp.maximum(cur, hc_in[kk, 0, 0])

        scale = cur / 448.0
        y = out_ref[...]
        q = (y / scale).astype(jnp.float8_e4m3fn)
        out_ref[...] = q.astype(jnp.float32) * scale

    return pl.pallas_call(
        body,
        out_shape=jax.ShapeDtypeStruct((m_total, n_local), jnp.float32),
        in_specs=[
            pl.BlockSpec(memory_space=pltpu.VMEM),
            pl.BlockSpec(memory_space=pltpu.VMEM),
        ],
        out_specs=pl.BlockSpec(memory_space=pltpu.VMEM),
        scratch_shapes=[
            pltpu.VMEM((N_DEV, m_per, k), jnp.bfloat16),
            pltpu.SemaphoreType.DMA((R_STEPS,)),
            pltpu.SemaphoreType.DMA((R_STEPS,)),
            pltpu.SemaphoreType.DMA((L_STEPS,)),
            pltpu.SemaphoreType.DMA((L_STEPS,)),
            pltpu.VMEM((1, 1), jnp.float32),
            pltpu.VMEM((HC_ROUNDS, 1, 1), jnp.float32),
            pltpu.SemaphoreType.DMA((HC_ROUNDS,)),
            pltpu.SemaphoreType.DMA((HC_ROUNDS,)),
        ],
        compiler_params=pltpu.CompilerParams(collective_id=0),
    )(x, w_mat)


# baseline (device time: 427091 ns/iter reference)
import jax
import jax.numpy as jnp
from jax import lax
from jax.experimental import pallas as pl
from jax.experimental.pallas import tpu as pltpu

N_DEV = 32
R_STEPS = N_DEV // 2
L_STEPS = N_DEV // 2 - 1
HC_ROUNDS = 5


def kernel(x, w_mat):
    x = x.astype(jnp.bfloat16)
    w_mat = w_mat.astype(jnp.bfloat16)
    m_per, k = x.shape
    _, n_local = w_mat.shape
    m_total = N_DEV * m_per

    def body(x_ref, w_ref, out_ref, comm_ref,
             r_send, r_recv, l_send, l_recv,
             hc_buf, hc_in, hc_ss, hc_rs):
        my = lax.axis_index("i")
        right = lax.rem(my + 1, N_DEV)
        left = lax.rem(my + N_DEV - 1, N_DEV)

        barrier = pltpu.get_barrier_semaphore()
        for nbr in (left, right):
            pl.semaphore_signal(barrier, inc=1, device_id=(nbr,),
                                device_id_type=pl.DeviceIdType.MESH)
        pl.semaphore_wait(barrier, 2)

        comm_ref[0] = x_ref[...]

        def gemm(slot):
            origin = lax.rem(my - slot + N_DEV, N_DEV)
            out_ref[pl.ds(origin * m_per, m_per), :] = jnp.dot(
                comm_ref[slot], w_ref[...], preferred_element_type=jnp.float32)

        rdmas = []
        for r in range(1, R_STEPS + 1):
            rd = pltpu.make_async_remote_copy(
                src_ref=comm_ref.at[r - 1], dst_ref=comm_ref.at[r],
                send_sem=r_send.at[r - 1], recv_sem=r_recv.at[r - 1],
                device_id=(right,), device_id_type=pl.DeviceIdType.MESH)
            rd.start()
            rdmas.append(rd)
            ld = None
            if r <= L_STEPS:
                src_slot = 0 if r == 1 else N_DEV + 1 - r
                ld = pltpu.make_async_remote_copy(
                    src_ref=comm_ref.at[src_slot], dst_ref=comm_ref.at[N_DEV - r],
                    send_sem=l_send.at[r - 1], recv_sem=l_recv.at[r - 1],
                    device_id=(left,), device_id_type=pl.DeviceIdType.MESH)
                ld.start()
                rdmas.append(ld)
            if r == 1:
                gemm(0)
            else:
                gemm(r - 1)
                gemm(N_DEV + 1 - r)
            rd.wait_recv()
            if ld is not None:
                ld.wait_recv()
        gemm(R_STEPS)
        gemm(R_STEPS + 1)
        for rd in rdmas:
            rd.wait_send()

        cur = jnp.max(jnp.abs(out_ref[...]))
        for kk in range(HC_ROUNDS):
            partner = jnp.bitwise_xor(my, 1 << kk)
            hc_buf[...] = cur[None, None]
            rd = pltpu.make_async_remote_copy(
                src_ref=hc_buf, dst_ref=hc_in.at[kk],
                send_sem=hc_ss.at[kk], recv_sem=hc_rs.at[kk],
                device_id=(partner,), device_id_type=pl.DeviceIdType.MESH)
            rd.start()
            rd.wait()
            cur = jnp.maximum(cur, jnp.max(hc_in[kk]))

        scale = cur / 448.0
        y = out_ref[...]
        q = (y / scale).astype(jnp.float8_e4m3fn)
        out_ref[...] = q.astype(jnp.float32) * scale

    return pl.pallas_call(
        body,
        out_shape=jax.ShapeDtypeStruct((m_total, n_local), jnp.float32),
        in_specs=[
            pl.BlockSpec(memory_space=pltpu.VMEM),
            pl.BlockSpec(memory_space=pltpu.VMEM),
        ],
        out_specs=pl.BlockSpec(memory_space=pltpu.VMEM),
        scratch_shapes=[
            pltpu.VMEM((N_DEV, m_per, k), jnp.bfloat16),
            pltpu.SemaphoreType.DMA((R_STEPS,)),
            pltpu.SemaphoreType.DMA((R_STEPS,)),
            pltpu.SemaphoreType.DMA((L_STEPS,)),
            pltpu.SemaphoreType.DMA((L_STEPS,)),
            pltpu.VMEM((1, 1), jnp.float32),
            pltpu.VMEM((HC_ROUNDS, 1, 1), jnp.float32),
            pltpu.SemaphoreType.DMA((HC_ROUNDS,)),
            pltpu.SemaphoreType.DMA((HC_ROUNDS,)),
        ],
        compiler_params=pltpu.CompilerParams(
            collective_id=0, vmem_limit_bytes=60 * 1024 * 1024),
    )(x, w_mat)


# device time: 242513 ns/iter; 1.7611x vs baseline; 1.7611x over previous
import jax
import jax.numpy as jnp
from jax import lax
from jax.experimental import pallas as pl
from jax.experimental.pallas import tpu as pltpu

N_DEV = 32
R_STEPS = N_DEV // 2
L_STEPS = N_DEV // 2 - 1
HC_ROUNDS = 5


def _cycle_logical_ids():
    devs = [d for d in jax.devices()
            if getattr(d, "core_on_chip", 1) == 1]
    coords = sorted(tuple(d.coords) for d in devs)
    if len(coords) != N_DEV:
        return list(range(N_DEV))
    x0 = min(c[0] for c in coords)
    y0 = min(c[1] for c in coords)
    z0 = min(c[2] for c in coords)
    norm = [(c[0] - x0, c[1] - y0, c[2] - z0) for c in coords]
    if set(norm) != {(x, y, z) for x in range(2) for y in range(4)
                     for z in range(4)}:
        return list(range(N_DEV))
    plane = [(0, 0), (1, 0), (1, 1), (0, 1), (0, 2), (1, 2), (1, 3), (0, 3)]
    logical_of = {}
    lid = 0
    for z in range(4):
        for (x, y) in plane:
            logical_of[(x, y, z)] = lid
            lid += 1
    path = []
    for z in range(4):
        ys = range(4) if z % 2 == 0 else range(3, -1, -1)
        path.extend((y, z) for y in ys)
    cycle = [(0, y, z) for (y, z) in path] + \
            [(1, y, z) for (y, z) in reversed(path)]
    return [logical_of[c] for c in cycle]


def kernel(x, w_mat):
    x = x.astype(jnp.bfloat16)
    w_mat = w_mat.astype(jnp.bfloat16)
    m_per, k = x.shape
    _, n_local = w_mat.shape
    m_total = N_DEV * m_per

    cycle = jnp.asarray(_cycle_logical_ids(), dtype=jnp.int32)
    pos = jnp.zeros((N_DEV,), jnp.int32).at[cycle].set(
        jnp.arange(N_DEV, dtype=jnp.int32))
    my_cyc = jnp.take(pos, lax.axis_index("i"))
    origin_ids = jnp.take(
        cycle, jnp.mod(my_cyc - jnp.arange(N_DEV, dtype=jnp.int32), N_DEV))

    def body(ids_ref, x_ref, w_ref, out_ref, comm_ref,
             r_send, r_recv, l_send, l_recv,
             hc_buf, hc_in, hc_ss, hc_rs):
        my = lax.axis_index("i")
        left = ids_ref[1]
        right = ids_ref[N_DEV - 1]

        barrier = pltpu.get_barrier_semaphore()
        for nbr in (left, right):
            pl.semaphore_signal(barrier, inc=1, device_id=(nbr,),
                                device_id_type=pl.DeviceIdType.MESH)
        pl.semaphore_wait(barrier, 2)

        comm_ref[0] = x_ref[...]

        def gemm(slot):
            origin = ids_ref[slot]
            out_ref[pl.ds(origin * m_per, m_per), :] = jnp.dot(
                comm_ref[slot], w_ref[...], preferred_element_type=jnp.float32)

        rdmas = []
        for r in range(1, R_STEPS + 1):
            rd = pltpu.make_async_remote_copy(
                src_ref=comm_ref.at[r - 1], dst_ref=comm_ref.at[r],
                send_sem=r_send.at[r - 1], recv_sem=r_recv.at[r - 1],
                device_id=(right,), device_id_type=pl.DeviceIdType.MESH)
            rd.start()
            rdmas.append(rd)
            ld = None
            if r <= L_STEPS:
                src_slot = 0 if r == 1 else N_DEV + 1 - r
                ld = pltpu.make_async_remote_copy(
                    src_ref=comm_ref.at[src_slot], dst_ref=comm_ref.at[N_DEV - r],
                    send_sem=l_send.at[r - 1], recv_sem=l_recv.at[r - 1],
                    device_id=(left,), device_id_type=pl.DeviceIdType.MESH)
                ld.start()
                rdmas.append(ld)
            if r == 1:
                gemm(0)
            else:
                gemm(r - 1)
                gemm(N_DEV + 1 - r)
            rd.wait_recv()
            if ld is not None:
                ld.wait_recv()
        gemm(R_STEPS)
        gemm(R_STEPS + 1)
        for rd in rdmas:
            rd.wait_send()

        cur = jnp.max(jnp.abs(out_ref[...]))
        for kk in range(HC_ROUNDS):
            partner = jnp.bitwise_xor(my, 1 << kk)
            hc_buf[...] = cur[None, None]
            rd = pltpu.make_async_remote_copy(
                src_ref=hc_buf, dst_ref=hc_in.at[kk],
                send_sem=hc_ss.at[kk], recv_sem=hc_rs.at[kk],
                device_id=(partner,), device_id_type=pl.DeviceIdType.MESH)
            rd.start()
            rd.wait()
            cur = jnp.maximum(cur, jnp.max(hc_in[kk]))

        scale = cur / 448.0
        y = out_ref[...]
        q = (y / scale).astype(jnp.float8_e4m3fn)
        out_ref[...] = q.astype(jnp.float32) * scale

    return pl.pallas_call(
        body,
        out_shape=jax.ShapeDtypeStruct((m_total, n_local), jnp.float32),
        in_specs=[
            pl.BlockSpec(memory_space=pltpu.SMEM),
            pl.BlockSpec(memory_space=pltpu.VMEM),
            pl.BlockSpec(memory_space=pltpu.VMEM),
        ],
        out_specs=pl.BlockSpec(memory_space=pltpu.VMEM),
        scratch_shapes=[
            pltpu.VMEM((N_DEV, m_per, k), jnp.bfloat16),
            pltpu.SemaphoreType.DMA((R_STEPS,)),
            pltpu.SemaphoreType.DMA((R_STEPS,)),
            pltpu.SemaphoreType.DMA((L_STEPS,)),
            pltpu.SemaphoreType.DMA((L_STEPS,)),
            pltpu.VMEM((1, 1), jnp.float32),
            pltpu.VMEM((HC_ROUNDS, 1, 1), jnp.float32),
            pltpu.SemaphoreType.DMA((HC_ROUNDS,)),
            pltpu.SemaphoreType.DMA((HC_ROUNDS,)),
        ],
        compiler_params=pltpu.CompilerParams(
            collective_id=0, vmem_limit_bytes=60 * 1024 * 1024),
    )(origin_ids, x, w_mat)
